# baseline (speedup 1.0000x reference)
import functools

import jax
import jax.numpy as jnp
from jax import lax
from jax.experimental import pallas as pl
from jax.experimental.pallas import tpu as pltpu
from jax.experimental.pallas import tpu_sc as plsc

_N_SUB = 1_000_000
_N_IDS = 16384
_NC = 2
_NS = 16
_NW = _NC * _NS
_IDS_PER_W = _N_IDS // _NW


def _zero_body(out_ref):
    out_ref[...] = jnp.zeros((_N_SUB,), jnp.float32)


def _zeros_tc():
    return pl.pallas_call(
        _zero_body,
        out_shape=jax.ShapeDtypeStruct((_N_SUB,), jnp.float32),
    )()


_mesh = plsc.VectorSubcoreMesh(core_axis_name="c", subcore_axis_name="s")


@functools.partial(
    pl.kernel,
    mesh=_mesh,
    scratch_types=[
        pltpu.VMEM((_IDS_PER_W,), jnp.int32),
        pltpu.VMEM((16,), jnp.float32),
        pltpu.SemaphoreType.DMA,
    ],
)
def _scatter_kernel(ids_hbm, out_ref, idx_v, ones_v, sem):
    wid = lax.axis_index("s") * _NC + lax.axis_index("c")
    base = wid * _IDS_PER_W

    pltpu.sync_copy(ids_hbm.at[pl.ds(base, _IDS_PER_W)], idx_v)
    ones_v[...] = jnp.full((16,), 1.0, jnp.float32)

    def body(i, carry):
        idx16 = idx_v[pl.ds(i * 16, 16)]
        pltpu.async_copy(ones_v, out_ref.at[idx16], sem)
        return carry

    lax.fori_loop(0, _IDS_PER_W // 16, body, 0)

    def drain(i, carry):
        pltpu.make_async_copy(ones_v, out_ref.at[idx_v[pl.ds(i * 16, 16)]], sem).wait()
        return carry

    lax.fori_loop(0, _IDS_PER_W // 16, drain, 0)


def kernel(subject_ids):
    ids = jnp.reshape(subject_ids, (-1,)).astype(jnp.int32)
    out = jax.new_ref(_zeros_tc())
    _scatter_kernel(ids, out)
    return out[...]


# speedup vs baseline: 1.0050x; 1.0050x over previous
import functools

import jax
import jax.numpy as jnp
from jax import lax
from jax.experimental import pallas as pl
from jax.experimental.pallas import tpu as pltpu
from jax.experimental.pallas import tpu_sc as plsc

_N_SUB = 1_000_000
_N_IDS = 16384
_NC = 2
_NS = 16
_NW = _NC * _NS
_ROWS = 4
_COLS = 128


def _zero_body(out_ref):
    out_ref[...] = jnp.zeros((_N_SUB,), jnp.float32)


def _zeros_tc():
    return pl.pallas_call(
        _zero_body,
        out_shape=jax.ShapeDtypeStruct((_N_SUB,), jnp.float32),
    )()


_mesh = plsc.VectorSubcoreMesh(core_axis_name="c", subcore_axis_name="s")


@functools.partial(
    pl.kernel,
    mesh=_mesh,
    scratch_types=[
        pltpu.VMEM((_ROWS, _COLS), jnp.int32),
        pltpu.VMEM((_COLS,), jnp.float32),
        pltpu.SemaphoreType.DMA,
    ],
)
def _scatter_kernel(ids_hbm, out_ref, idx_v, ones_v, sem):
    wid = lax.axis_index("s") * _NC + lax.axis_index("c")

    pltpu.sync_copy(ids_hbm.at[wid], idx_v)

    one16 = jnp.full((16,), 1.0, jnp.float32)
    for j in range(_COLS // 16):
        ones_v[pl.ds(j * 16, 16)] = one16

    for j in range(_ROWS):
        pltpu.async_copy(ones_v, out_ref.at[idx_v.at[j]], sem)
    for j in range(_ROWS):
        pltpu.make_async_copy(ones_v, out_ref.at[idx_v.at[j]], sem).wait()


def kernel(subject_ids):
    ids = jnp.reshape(subject_ids, (_NW, _ROWS, _COLS)).astype(jnp.int32)
    out = jax.new_ref(_zeros_tc())
    _scatter_kernel(ids, out)
    return out[...]


# speedup vs baseline: 1.4490x; 1.4418x over previous
import functools

import jax
import jax.numpy as jnp
from jax import lax
from jax.experimental import pallas as pl
from jax.experimental.pallas import tpu as pltpu
from jax.experimental.pallas import tpu_sc as plsc

_N_SUB = 1_000_000
_N_IDS = 16384
_NC = 2
_NS = 16
_NW = _NC * _NS
_CHUNK = 31248
_LAST = _N_SUB - (_NW - 1) * _CHUNK
_SCRATCH = 31488

_mesh = plsc.VectorSubcoreMesh(core_axis_name="c", subcore_axis_name="s")


@functools.partial(
    pl.kernel,
    out_type=jax.ShapeDtypeStruct((_N_SUB,), jnp.float32),
    mesh=_mesh,
    scratch_types=[
        pltpu.VMEM((_N_IDS,), jnp.int32),
        pltpu.VMEM((_SCRATCH,), jnp.float32),
        pltpu.SemaphoreType.DMA,
    ],
    compiler_params=pltpu.CompilerParams(
        needs_layout_passes=False,
        disable_bounds_checks=True,
        disable_semaphore_checks=True,
    ),
)
def _mask_kernel(ids_hbm, out_hbm, ids_v, chunk_v, sem):
    wid = lax.axis_index("s") * _NC + lax.axis_index("c")
    lo = wid * _CHUNK
    hi = jnp.where(wid == _NW - 1, _N_SUB, lo + _CHUNK)

    ids_copy = pltpu.async_copy(ids_hbm, ids_v, sem)

    zero16 = jnp.zeros((16,), jnp.float32)

    @plsc.parallel_loop(0, _SCRATCH, step=256, unroll=2)
    def _(base):
        for j in range(16):
            chunk_v[pl.ds(base + j * 16, 16)] = zero16

    ids_copy.wait()

    ones16 = jnp.full((16,), 1.0, jnp.float32)
    size_u = lax.convert_element_type(hi - lo, jnp.uint32)

    @plsc.parallel_loop(0, _N_IDS, step=128, unroll=4)
    def _(base):
        for j in range(8):
            ids16 = ids_v[pl.ds(base + j * 16, 16)]
            local = ids16 - lo
            inb = plsc.bitcast(local, jnp.uint32) < size_u
            plsc.store_scatter(chunk_v, [local], ones16, mask=inb)

    @pl.when(wid < _NW - 1)
    def _():
        pltpu.sync_copy(chunk_v.at[pl.ds(0, _CHUNK)], out_hbm.at[pl.ds(lo, _CHUNK)])

    @pl.when(wid == _NW - 1)
    def _():
        pltpu.sync_copy(
            chunk_v.at[pl.ds(0, _LAST)],
            out_hbm.at[pl.ds((_NW - 1) * _CHUNK, _LAST)],
        )


def kernel(subject_ids):
    ids = jnp.reshape(subject_ids, (-1,)).astype(jnp.int32)
    return _mask_kernel(ids)
